# Initial kernel scaffold; baseline (speedup 1.0000x reference)
import jax
import jax.numpy as jnp
from jax.experimental import pallas as pl


def kernel(x):
    raise NotImplementedError("write your pallas kernel here")


# speedup vs baseline: 153.6875x; 153.6875x over previous
import jax
import jax.numpy as jnp
from jax import lax
from jax.experimental import pallas as pl

_CONF_THRES = 0.001
_IOU_THRES = 0.45
_NC = 4
_MAX_WH = 4096.0
_MAX_DET = 300

_N = 5000
_NPAD = 5120
_ROWS = 40
_LANES = 128


def _nms_body(x_ref, o_ref):
    def fld(c):
        return x_ref[0, c, :].reshape(_ROWS, _LANES)

    cx, cy, w, h = fld(0), fld(1), fld(2), fld(3)
    obj = fld(4)
    x1 = cx - w / 2.0
    y1 = cy - h / 2.0
    x2 = cx + w / 2.0
    y2 = cy + h / 2.0

    c0 = fld(5) * obj
    c1 = fld(6) * obj
    c2 = fld(7) * obj
    c3 = fld(8) * obj
    conf = jnp.maximum(jnp.maximum(c0, c1), jnp.maximum(c2, c3))
    jf = jnp.where(
        c0 == conf,
        0.0,
        jnp.where(c1 == conf, 1.0, jnp.where(c2 == conf, 2.0, 3.0)),
    )

    idx = (
        lax.broadcasted_iota(jnp.int32, (_ROWS, _LANES), 0) * _LANES
        + lax.broadcasted_iota(jnp.int32, (_ROWS, _LANES), 1)
    )
    pad = idx >= _N
    scores0 = jnp.where((conf > _CONF_THRES) & (~pad), conf, -1.0)

    off = jf * _MAX_WH
    x1o = x1 + off
    y1o = y1 + off
    x2o = x2 + off
    y2o = y2 + off
    areao = (x2o - x1o) * (y2o - y1o)

    neg = jnp.float32(-jnp.inf)
    big = jnp.int32(1 << 30)

    def it(t, scores):
        s = jnp.max(scores)
        m = jnp.min(jnp.where(scores == s, idx, big))
        sel = idx == m

        def ext(arr):
            return jnp.sum(jnp.where(sel, arr, 0.0))

        bx1o = ext(x1o)
        by1o = ext(y1o)
        bx2o = ext(x2o)
        by2o = ext(y2o)
        barea = ext(areao)

        ltx = jnp.maximum(bx1o, x1o)
        lty = jnp.maximum(by1o, y1o)
        rbx = jnp.minimum(bx2o, x2o)
        rby = jnp.minimum(by2o, y2o)
        iw = jnp.clip(rbx - ltx, 0.0, None)
        ih = jnp.clip(rby - lty, 0.0, None)
        inter = iw * ih
        iou = inter / (barea + areao - inter + 1e-9)

        emit = s > _CONF_THRES
        kill = (iou > _IOU_THRES) | sel
        new_scores = jnp.where(jnp.logical_and(emit, kill), neg, scores)

        lane = lax.broadcasted_iota(jnp.int32, (1, _LANES), 1)
        row = jnp.where(
            lane == 0,
            ext(x1),
            jnp.where(
                lane == 1,
                ext(y1),
                jnp.where(
                    lane == 2,
                    ext(x2),
                    jnp.where(
                        lane == 3,
                        ext(y2),
                        jnp.where(lane == 4, s, jnp.where(lane == 5, ext(jf), 0.0)),
                    ),
                ),
            ),
        )
        o_ref[0, pl.ds(t, 1), :] = jnp.where(emit, row, 0.0)
        return new_scores

    lax.fori_loop(0, _MAX_DET, it, scores0)


def kernel(x):
    pred = x[0]
    b = pred.shape[0]
    predt = jnp.transpose(pred, (0, 2, 1))
    predt = jnp.pad(predt, ((0, 0), (0, 0), (0, _NPAD - _N)))

    out = pl.pallas_call(
        _nms_body,
        grid=(b,),
        in_specs=[
            pl.BlockSpec((1, predt.shape[1], _NPAD), lambda i: (i, 0, 0)),
        ],
        out_specs=pl.BlockSpec((1, _MAX_DET, _LANES), lambda i: (i, 0, 0)),
        out_shape=jax.ShapeDtypeStruct((b, _MAX_DET, _LANES), jnp.float32),
    )(predt)
    return out[:, :, :6]


# speedup vs baseline: 173.9017x; 1.1315x over previous
import jax
import jax.numpy as jnp
from jax import lax
from jax.experimental import pallas as pl

_CONF_THRES = 0.001
_IOU_THRES = 0.45
_NC = 4
_MAX_WH = 4096.0
_MAX_DET = 300

_N = 5000
_NPAD = 5120
_ROWS = 40
_LANES = 128


def _nms_body(x_ref, o_ref):
    def fld(c):
        return x_ref[:, c, :].reshape(-1, _ROWS, _LANES)

    cx, cy, w, h = fld(0), fld(1), fld(2), fld(3)
    obj = fld(4)
    x1 = cx - w / 2.0
    y1 = cy - h / 2.0
    x2 = cx + w / 2.0
    y2 = cy + h / 2.0

    c0 = fld(5) * obj
    c1 = fld(6) * obj
    c2 = fld(7) * obj
    c3 = fld(8) * obj
    conf = jnp.maximum(jnp.maximum(c0, c1), jnp.maximum(c2, c3))
    jf = jnp.where(
        c0 == conf,
        0.0,
        jnp.where(c1 == conf, 1.0, jnp.where(c2 == conf, 2.0, 3.0)),
    )

    idx = (
        lax.broadcasted_iota(jnp.int32, (_ROWS, _LANES), 0) * _LANES
        + lax.broadcasted_iota(jnp.int32, (_ROWS, _LANES), 1)
    )[None]
    pad = idx >= _N
    scores0 = jnp.where((conf > _CONF_THRES) & (~pad), conf, -1.0)

    off = jf * _MAX_WH
    x1o = x1 + off
    y1o = y1 + off
    x2o = x2 + off
    y2o = y2 + off
    areao = (x2o - x1o) * (y2o - y1o)

    neg = jnp.float32(-jnp.inf)
    big = jnp.int32(1 << 30)

    def bb(v):
        return v[:, None, None]

    def it(t, scores):
        s = jnp.max(scores, axis=(1, 2))
        m = jnp.min(jnp.where(scores == bb(s), idx, big), axis=(1, 2))
        sel = idx == bb(m)

        def ext(arr):
            return jnp.sum(jnp.where(sel, arr, 0.0), axis=(1, 2))

        bx1o = ext(x1o)
        by1o = ext(y1o)
        bx2o = ext(x2o)
        by2o = ext(y2o)
        barea = ext(areao)

        ltx = jnp.maximum(bb(bx1o), x1o)
        lty = jnp.maximum(bb(by1o), y1o)
        rbx = jnp.minimum(bb(bx2o), x2o)
        rby = jnp.minimum(bb(by2o), y2o)
        iw = jnp.clip(rbx - ltx, 0.0, None)
        ih = jnp.clip(rby - lty, 0.0, None)
        inter = iw * ih
        iou = inter / (bb(barea) + areao - inter + 1e-9)

        emit = s > _CONF_THRES
        kill = (iou > _IOU_THRES) | sel
        new_scores = jnp.where(jnp.logical_and(bb(emit), kill), neg, scores)

        lane = lax.broadcasted_iota(jnp.int32, (1, 1, _LANES), 2)
        row = jnp.where(
            lane == 0,
            bb(ext(x1)),
            jnp.where(
                lane == 1,
                bb(ext(y1)),
                jnp.where(
                    lane == 2,
                    bb(ext(x2)),
                    jnp.where(
                        lane == 3,
                        bb(ext(y2)),
                        jnp.where(
                            lane == 4,
                            bb(s),
                            jnp.where(lane == 5, bb(ext(jf)), 0.0),
                        ),
                    ),
                ),
            ),
        )
        o_ref[:, pl.ds(t, 1), :] = jnp.where(bb(emit), row, 0.0)
        return new_scores

    lax.fori_loop(0, _MAX_DET, it, scores0)


def kernel(x):
    pred = x[0]
    b = pred.shape[0]
    predt = jnp.transpose(pred, (0, 2, 1))
    predt = jnp.pad(predt, ((0, 0), (0, 0), (0, _NPAD - _N)))

    out = pl.pallas_call(
        _nms_body,
        out_shape=jax.ShapeDtypeStruct((b, _MAX_DET, _LANES), jnp.float32),
    )(predt)
    return out[:, :, :6]
